# baseline (speedup 1.0000x reference)
import functools

import jax
import jax.numpy as jnp
from jax import lax
from jax.experimental import pallas as pl
from jax.experimental.pallas import tpu as pltpu
from jax.experimental.pallas import tpu_sc as plsc

N_NODES = 10000
D_FEAT = 128
N_EDGES = 320000

NC = 2
NS = 16
NVREG = D_FEAT // 16
CHUNK = 64
NCHUNK = 160
NB = 8
NBLK = NCHUNK // NB
NBUF = 4
NRND = NB // NBUF
EPT = NCHUNK * CHUNK
E_PAD = NC * NS * EPT
GPB = 16
NGRP = CHUNK // GPB
N_PAD = 10240
RPT = N_PAD // NS
ZB = RPT // CHUNK

_mesh = plsc.VectorSubcoreMesh(core_axis_name="c", subcore_axis_name="s")


@functools.partial(
    pl.kernel,
    out_type=jax.ShapeDtypeStruct((NC, N_PAD, D_FEAT), jnp.float32),
    mesh=_mesh,
    scratch_types=[
        pltpu.VMEM_SHARED((N_PAD, D_FEAT), jnp.float32),
        pltpu.VMEM((NCHUNK // 2, 2 * CHUNK), jnp.int32),
        pltpu.VMEM((2, NB, CHUNK), jnp.int32),
        pltpu.VMEM((2, NB, CHUNK), jnp.float32),
        pltpu.VMEM((CHUNK, D_FEAT), jnp.float32),
        pltpu.VMEM((CHUNK, D_FEAT), jnp.float32),
        pltpu.VMEM((CHUNK, D_FEAT), jnp.float32),
        pltpu.VMEM((CHUNK, D_FEAT), jnp.float32),
        pltpu.SemaphoreType.DMA,
        pltpu.SemaphoreType.DMA,
        pltpu.SemaphoreType.DMA,
        pltpu.SemaphoreType.DMA,
        pltpu.SemaphoreType.DMA,
        pltpu.SemaphoreType.DMA,
        pltpu.SemaphoreType.DMA,
        pltpu.SemaphoreType.DMA,
        pltpu.SemaphoreType.DMA,
    ],
)
def _mp_sc_kernel(x_hbm, idxj_hbm, idxi_hbm, val_hbm, out_hbm,
                  acc, idxj_v, idxi_v, val_v,
                  rows0, rows1, rows2, rows3,
                  g0, g1, g2, g3, s0, s1, s2, s3, isem):
    c = lax.axis_index("c")
    s = lax.axis_index("s")
    row0 = s * RPT
    bufs = (rows0, rows1, rows2, rows3)
    gsems = (g0, g1, g2, g3)
    ssems = (s0, s1, s2, s3)

    pltpu.sync_copy(idxj_hbm.at[c].at[s], idxj_v)
    pltpu.sync_copy(idxi_hbm.at[c].at[s].at[0], idxi_v.at[0])
    pltpu.sync_copy(val_hbm.at[c].at[s].at[0], val_v.at[0])

    zeros16 = jnp.zeros((16,), jnp.float32)

    def zero_row(r, carry):
        for q in range(NVREG):
            rows0[r, pl.ds(q * 16, 16)] = zeros16
        return carry

    lax.fori_loop(0, CHUNK, zero_row, 0)
    for b in range(ZB):
        pltpu.async_copy(rows0, acc.at[pl.ds(row0 + b * CHUNK, CHUNK)], isem)
    for b in range(ZB):
        pltpu.make_async_copy(
            rows0, acc.at[pl.ds(row0 + b * CHUNK, CHUNK)], isem).wait()
    plsc.subcore_barrier()

    def scale_rows(rows_ref, slot, l):
        def group_body(g, carry2):
            v16 = val_v[slot, l, pl.ds(g * GPB, GPB)]
            for r in range(GPB):
                vvec = jnp.full((16,), v16[r], jnp.float32)
                row = g * GPB + r
                for q in range(NVREG):
                    rows_ref[row, pl.ds(q * 16, 16)] = (
                        rows_ref[row, pl.ds(q * 16, 16)] * vvec)
            return carry2

        lax.fori_loop(0, NGRP, group_body, 0)

    def issue_gather(k, u):
        pltpu.async_copy(
            x_hbm.at[idxj_v.at[k // 2, pl.ds((k % 2) * CHUNK, CHUNK)]],
            bufs[u], gsems[u])

    def wait_gather(u):
        pltpu.make_async_copy(x_hbm.at[pl.ds(0, CHUNK)], bufs[u],
                              gsems[u]).wait()

    def issue_scatter(slot, l, u):
        pltpu.async_copy(bufs[u], acc.at[idxi_v.at[slot].at[l]], ssems[u],
                         add=True)

    def wait_scatter(u):
        pltpu.make_async_copy(bufs[u], acc.at[pl.ds(0, CHUNK)],
                              ssems[u]).wait()

    for u in range(NBUF):
        issue_gather(u, u)

    def block_body(b, carry):
        slot = lax.rem(b, 2)
        nslot = lax.rem(b + 1, 2)

        @pl.when(b > 0)
        def _():
            pltpu.make_async_copy(idxi_hbm.at[c].at[s].at[b],
                                  idxi_v.at[slot], isem).wait()
            pltpu.make_async_copy(val_hbm.at[c].at[s].at[b],
                                  val_v.at[slot], isem).wait()

        for r in range(NRND):
            for u in range(NBUF):
                l = r * NBUF + u
                wait_gather(u)
                scale_rows(bufs[u], slot, l)
                issue_scatter(slot, l, u)

            if r == 0:
                @pl.when(b + 1 < NBLK)
                def _():
                    pltpu.async_copy(idxi_hbm.at[c].at[s].at[b + 1],
                                     idxi_v.at[nslot], isem)
                    pltpu.async_copy(val_hbm.at[c].at[s].at[b + 1],
                                     val_v.at[nslot], isem)

            for u in range(NBUF):
                k = b * NB + r * NBUF + u
                wait_scatter(u)

                @pl.when(k + NBUF < NCHUNK)
                def _():
                    issue_gather(k + NBUF, u)
        return carry

    lax.fori_loop(0, NBLK, block_body, 0)
    plsc.subcore_barrier()

    pltpu.sync_copy(acc.at[pl.ds(row0, RPT)], out_hbm.at[c].at[pl.ds(row0, RPT)])


def _combine_body(p_ref, o_ref):
    o_ref[...] = p_ref[0] + p_ref[1]


_N_BLK = 8


def _combine(partials):
    return pl.pallas_call(
        _combine_body,
        out_shape=jax.ShapeDtypeStruct((N_PAD, D_FEAT), jnp.float32),
        grid=(_N_BLK,),
        in_specs=[pl.BlockSpec((NC, N_PAD // _N_BLK, D_FEAT),
                               lambda i: (0, i, 0))],
        out_specs=pl.BlockSpec((N_PAD // _N_BLK, D_FEAT), lambda i: (i, 0)),
    )(partials)


def kernel(x, a_indices, a_values):
    pad = E_PAD - N_EDGES
    idx_i = jnp.pad(a_indices[0].astype(jnp.int32), (0, pad))
    idx_j = jnp.pad(a_indices[1].astype(jnp.int32), (0, pad))
    vals = jnp.pad(a_values.astype(jnp.float32), (0, pad))
    idx_i = idx_i.reshape(NC, NS, NBLK, NB, CHUNK)
    idx_j = idx_j.reshape(NC, NS, NCHUNK // 2, 2 * CHUNK)
    vals = vals.reshape(NC, NS, NBLK, NB, CHUNK)
    x_pad = jnp.pad(x, ((0, N_PAD - N_NODES), (0, 0)))
    partials = _mp_sc_kernel(x_pad, idx_j, idx_i, vals)
    return _combine(partials)[:N_NODES]
